# Initial kernel scaffold; baseline (speedup 1.0000x reference)
import jax
import jax.numpy as jnp
from jax.experimental import pallas as pl


def kernel(relative_position_bias_table, relative_position_index):
    raise NotImplementedError("write your pallas kernel here")


# speedup vs baseline: 3.4675x; 3.4675x over previous
import functools

import jax
import jax.numpy as jnp
from jax import lax
from jax.experimental import pallas as pl
from jax.experimental.pallas import tpu as pltpu
from jax.experimental.pallas import tpu_sc as plsc

_N = 576 * 576
_H = 32
_ROWS = 2209
_NC, _NS, _L = 2, 16, 16
_NW = _NC * _NS
_PER_W = _N // _NW
_SUB = 1152
_NSUB = _PER_W // _SUB
_NV = _SUB // _L


def _make_kernel():
    mesh = plsc.VectorSubcoreMesh(core_axis_name="c", subcore_axis_name="s")

    @functools.partial(
        pl.kernel,
        mesh=mesh,
        out_type=jax.ShapeDtypeStruct((_H, _N), jnp.float32),
        scratch_types=[
            pltpu.VMEM((_ROWS * _H,), jnp.float32),
            pltpu.VMEM((_SUB,), jnp.int32),
            pltpu.VMEM((_H, _SUB), jnp.float32),
        ],
        compiler_params=pltpu.CompilerParams(
            use_tc_tiling_on_sc=False, needs_layout_passes=False
        ),
    )
    def k(table_hbm, idx_hbm, out_hbm, table_v, idx_v, out_v):
        wid = lax.axis_index("s") * _NC + lax.axis_index("c")
        pltpu.sync_copy(table_hbm, table_v)
        base = wid * _PER_W

        def sub_body(s, carry):
            off = base + s * _SUB
            pltpu.sync_copy(idx_hbm.at[pl.ds(off, _SUB)], idx_v)

            def v_body(v, c2):
                iv = idx_v[pl.ds(v * _L, _L)]
                iv_flat = iv * _H
                for h in range(_H):
                    vals = plsc.load_gather(table_v, [iv_flat + h])
                    out_v[h, pl.ds(v * _L, _L)] = vals
                return c2

            lax.fori_loop(0, _NV, v_body, 0)
            pltpu.sync_copy(out_v, out_hbm.at[:, pl.ds(off, _SUB)])
            return carry

        lax.fori_loop(0, _NSUB, sub_body, 0)

    return k


_gather_kernel = _make_kernel()


def kernel(relative_position_bias_table, relative_position_index):
    table_flat = relative_position_bias_table.reshape(-1)
    idx_flat = relative_position_index.reshape(-1).astype(jnp.int32)
    out = _gather_kernel(table_flat, idx_flat)
    n0, n1 = relative_position_index.shape
    return out.reshape(_H, n0, n1)


# speedup vs baseline: 5.2703x; 1.5199x over previous
import functools

import jax
import jax.numpy as jnp
from jax import lax
from jax.experimental import pallas as pl
from jax.experimental.pallas import tpu as pltpu
from jax.experimental.pallas import tpu_sc as plsc

_N = 576 * 576
_H = 32
_ROWS = 2209
_NC, _NS, _L = 2, 16, 16
_NW = _NC * _NS
_PER_W = _N // _NW
_SUB = 384
_NSUB = _PER_W // _SUB
_NV = _SUB // _L


def _make_kernel():
    mesh = plsc.VectorSubcoreMesh(core_axis_name="c", subcore_axis_name="s")

    @functools.partial(
        pl.kernel,
        mesh=mesh,
        out_type=jax.ShapeDtypeStruct((_H, _N), jnp.float32),
        scratch_types=[
            pltpu.VMEM((_ROWS * _H,), jnp.float32),
            pltpu.VMEM((_PER_W,), jnp.int32),
            pltpu.VMEM((2, _H, _SUB), jnp.float32),
            pltpu.SemaphoreType.DMA,
            pltpu.SemaphoreType.DMA,
            pltpu.SemaphoreType.DMA,
        ],
        compiler_params=pltpu.CompilerParams(
            use_tc_tiling_on_sc=False, needs_layout_passes=False
        ),
    )
    def k(table_hbm, idx_hbm, out_hbm, table_v, idx_v, out_v, sem_in, sem0, sem1):
        wid = lax.axis_index("s") * _NC + lax.axis_index("c")
        base = wid * _PER_W

        cp_t = pltpu.make_async_copy(table_hbm, table_v, sem_in)
        cp_t.start()
        cp_i = pltpu.make_async_copy(idx_hbm.at[pl.ds(base, _PER_W)], idx_v, sem_in)
        cp_i.start()
        cp_t.wait()
        cp_i.wait()

        sems = (sem0, sem1)

        def gather_round(s, buf):
            off = s * _SUB

            @plsc.parallel_loop(0, _NV, unroll=2)
            def _(v):
                iv = idx_v[pl.ds(off + v * _L, _L)]
                iv_flat = iv * _H
                for h in range(_H):
                    out_v[buf, h, pl.ds(v * _L, _L)] = plsc.load_gather(
                        table_v, [iv_flat + h]
                    )

            pltpu.make_async_copy(
                out_v.at[buf], out_hbm.at[:, pl.ds(base + off, _SUB)], sems[buf]
            ).start()

        def wait_round(buf):
            pltpu.make_async_copy(
                out_v.at[buf], out_hbm.at[:, pl.ds(0, _SUB)], sems[buf]
            ).wait()

        gather_round(0, 0)
        gather_round(1, 1)

        def pair_body(p, carry):
            s = 2 + 2 * p
            wait_round(0)
            gather_round(s, 0)
            wait_round(1)
            gather_round(s + 1, 1)
            return carry

        lax.fori_loop(0, (_NSUB - 3) // 2, pair_body, 0)

        wait_round(0)
        gather_round(_NSUB - 1, 0)
        wait_round(1)
        wait_round(0)

    return k


_gather_kernel = _make_kernel()


def kernel(relative_position_bias_table, relative_position_index):
    table_flat = relative_position_bias_table.reshape(-1)
    idx_flat = relative_position_index.reshape(-1).astype(jnp.int32)
    out = _gather_kernel(table_flat, idx_flat)
    n0, n1 = relative_position_index.shape
    return out.reshape(_H, n0, n1)


# speedup vs baseline: 11.0140x; 2.0898x over previous
import functools

import jax
import jax.numpy as jnp
from jax import lax
from jax.experimental import pallas as pl
from jax.experimental.pallas import tpu as pltpu
from jax.experimental.pallas import tpu_sc as plsc

_N = 576 * 576
_H = 32
_ROWS = 2209
_NC, _NS, _L = 2, 16, 16
_NW = _NC * _NS
_PER_W = _N // _NW
_SUB = 384
_NSUB = _PER_W // _SUB
_NV = _SUB // _L


def _make_kernel():
    mesh = plsc.VectorSubcoreMesh(core_axis_name="c", subcore_axis_name="s")

    @functools.partial(
        pl.kernel,
        mesh=mesh,
        out_type=jax.ShapeDtypeStruct((_H, _N), jnp.float32),
        scratch_types=[
            pltpu.VMEM((_ROWS * _H,), jnp.float32),
            pltpu.VMEM((_PER_W,), jnp.int32),
            pltpu.VMEM((2, _H, _SUB), jnp.float32),
            pltpu.SemaphoreType.DMA,
            pltpu.SemaphoreType.DMA,
            pltpu.SemaphoreType.DMA,
        ],
        compiler_params=pltpu.CompilerParams(
            use_tc_tiling_on_sc=False, needs_layout_passes=False
        ),
    )
    def k(table_hbm, idx_hbm, out_hbm, table_v, idx_v, out_v, sem_in, sem0, sem1):
        wid = lax.axis_index("s") * _NC + lax.axis_index("c")
        base = wid * _PER_W

        cp_t = pltpu.make_async_copy(table_hbm, table_v, sem_in)
        cp_t.start()
        cp_i = pltpu.make_async_copy(idx_hbm.at[pl.ds(base, _PER_W)], idx_v, sem_in)
        cp_i.start()
        cp_t.wait()
        cp_i.wait()

        sems = (sem0, sem1)

        def gather_round(s, buf):
            off = s * _SUB

            @plsc.parallel_loop(0, _NV, unroll=2)
            def _(v):
                iv = idx_v[pl.ds(off + v * _L, _L)]
                for h in range(_H):
                    out_v[buf, h, pl.ds(v * _L, _L)] = plsc.load_gather(
                        table_v, [iv + h * _ROWS]
                    )

            pltpu.make_async_copy(
                out_v.at[buf], out_hbm.at[:, pl.ds(base + off, _SUB)], sems[buf]
            ).start()

        def wait_round(buf):
            pltpu.make_async_copy(
                out_v.at[buf], out_hbm.at[:, pl.ds(0, _SUB)], sems[buf]
            ).wait()

        gather_round(0, 0)
        gather_round(1, 1)

        def pair_body(p, carry):
            s = 2 + 2 * p
            wait_round(0)
            gather_round(s, 0)
            wait_round(1)
            gather_round(s + 1, 1)
            return carry

        lax.fori_loop(0, (_NSUB - 3) // 2, pair_body, 0)

        wait_round(0)
        gather_round(_NSUB - 1, 0)
        wait_round(1)
        wait_round(0)

    return k


_gather_kernel = _make_kernel()


def kernel(relative_position_bias_table, relative_position_index):
    table_flat = relative_position_bias_table.T.reshape(-1)
    idx_flat = relative_position_index.reshape(-1).astype(jnp.int32)
    out = _gather_kernel(table_flat, idx_flat)
    n0, n1 = relative_position_index.shape
    return out.reshape(_H, n0, n1)


# speedup vs baseline: 11.5984x; 1.0531x over previous
import functools

import jax
import jax.numpy as jnp
from jax import lax
from jax.experimental import pallas as pl
from jax.experimental.pallas import tpu as pltpu
from jax.experimental.pallas import tpu_sc as plsc

_N = 576 * 576
_H = 32
_ROWS = 2209
_NC, _NS, _L = 2, 16, 16
_NW = _NC * _NS
_PER_W = _N // _NW
_SUB = 384
_NSUB = _PER_W // _SUB
_NV = _SUB // _L


def _make_kernel():
    mesh = plsc.VectorSubcoreMesh(core_axis_name="c", subcore_axis_name="s")

    @functools.partial(
        pl.kernel,
        mesh=mesh,
        out_type=jax.ShapeDtypeStruct((_H, _N), jnp.float32),
        scratch_types=[
            pltpu.VMEM((_ROWS * _H,), jnp.float32),
            pltpu.VMEM((_PER_W,), jnp.int32),
            pltpu.VMEM((2, _H, _SUB), jnp.float32),
            pltpu.SemaphoreType.DMA,
            pltpu.SemaphoreType.DMA,
            pltpu.SemaphoreType.DMA,
        ],
        compiler_params=pltpu.CompilerParams(
            use_tc_tiling_on_sc=False, needs_layout_passes=False
        ),
    )
    def k(table_hbm, idx_hbm, out_hbm, table_v, idx_v, out_v, sem_in, sem0, sem1):
        wid = lax.axis_index("s") * _NC + lax.axis_index("c")
        base = wid * _PER_W

        cp_t = pltpu.make_async_copy(table_hbm, table_v, sem_in)
        cp_t.start()
        cp_i = pltpu.make_async_copy(idx_hbm.at[pl.ds(base, _PER_W)], idx_v, sem_in)
        cp_i.start()
        cp_t.wait()
        cp_i.wait()

        sems = (sem0, sem1)

        def gather_round(s, buf):
            off = s * _SUB

            @plsc.parallel_loop(0, _NV, unroll=2)
            def _(v):
                iv = idx_v[pl.ds(off + v * _L, _L)]

                @plsc.parallel_loop(0, _H, unroll=4)
                def _(h):
                    out_v[buf, h, pl.ds(v * _L, _L)] = plsc.load_gather(
                        table_v, [iv + h * _ROWS]
                    )

            pltpu.make_async_copy(
                out_v.at[buf], out_hbm.at[:, pl.ds(base + off, _SUB)], sems[buf]
            ).start()

        def wait_round(buf):
            pltpu.make_async_copy(
                out_v.at[buf], out_hbm.at[:, pl.ds(0, _SUB)], sems[buf]
            ).wait()

        def round_with_parity(s):
            @pl.when(s % 2 == 0)
            def _():
                gather_round(s, 0)

            @pl.when(s % 2 == 1)
            def _():
                gather_round(s, 1)

        def body(s, carry):
            @pl.when(s >= 2)
            def _():
                @pl.when(s % 2 == 0)
                def _():
                    wait_round(0)

                @pl.when(s % 2 == 1)
                def _():
                    wait_round(1)

            round_with_parity(s)
            return carry

        lax.fori_loop(0, _NSUB, body, 0)
        wait_round(1)
        wait_round(0)

    return k


_gather_kernel = _make_kernel()


def kernel(relative_position_bias_table, relative_position_index):
    table_flat = relative_position_bias_table.T.reshape(-1)
    idx_flat = relative_position_index.reshape(-1).astype(jnp.int32)
    out = _gather_kernel(table_flat, idx_flat)
    n0, n1 = relative_position_index.shape
    return out.reshape(_H, n0, n1)


# speedup vs baseline: 11.7019x; 1.0089x over previous
import functools

import jax
import jax.numpy as jnp
from jax import lax
from jax.experimental import pallas as pl
from jax.experimental.pallas import tpu as pltpu
from jax.experimental.pallas import tpu_sc as plsc

_N = 576 * 576
_H = 32
_ROWS = 2209
_NC, _NS, _L = 2, 16, 16
_NW = _NC * _NS
_PER_W = _N // _NW
_SUB = 384
_NSUB = _PER_W // _SUB
_NV = _SUB // _L


def _make_kernel():
    mesh = plsc.VectorSubcoreMesh(core_axis_name="c", subcore_axis_name="s")

    @functools.partial(
        pl.kernel,
        mesh=mesh,
        out_type=jax.ShapeDtypeStruct((_H, _N), jnp.float32),
        scratch_types=[
            pltpu.VMEM((_ROWS * _H,), jnp.float32),
            pltpu.VMEM((_PER_W,), jnp.int32),
            pltpu.VMEM((2, _H, _SUB), jnp.float32),
            pltpu.SemaphoreType.DMA,
            pltpu.SemaphoreType.DMA,
            pltpu.SemaphoreType.DMA,
        ],
        compiler_params=pltpu.CompilerParams(
            use_tc_tiling_on_sc=False, needs_layout_passes=False
        ),
    )
    def k(table_hbm, idx_hbm, out_hbm, table_v, idx_v, out_v, sem_in, sem0, sem1):
        wid = lax.axis_index("s") * _NC + lax.axis_index("c")
        base = wid * _PER_W

        cp_t = pltpu.make_async_copy(table_hbm, table_v, sem_in)
        cp_t.start()
        cp_i = pltpu.make_async_copy(idx_hbm.at[pl.ds(base, _PER_W)], idx_v, sem_in)
        cp_i.start()
        cp_t.wait()
        cp_i.wait()

        sems = (sem0, sem1)

        def gather_round(s, buf):
            off = s * _SUB

            @plsc.parallel_loop(0, _NV)
            def _(v):
                iv = idx_v[pl.ds(off + v * _L, _L)]

                @plsc.parallel_loop(0, _H, unroll=8)
                def _(h):
                    out_v[buf, h, pl.ds(v * _L, _L)] = plsc.load_gather(
                        table_v, [iv + h * _ROWS]
                    )

            pltpu.make_async_copy(
                out_v.at[buf], out_hbm.at[:, pl.ds(base + off, _SUB)], sems[buf]
            ).start()

        def wait_round(buf):
            pltpu.make_async_copy(
                out_v.at[buf], out_hbm.at[:, pl.ds(0, _SUB)], sems[buf]
            ).wait()

        def round_with_parity(s):
            @pl.when(s % 2 == 0)
            def _():
                gather_round(s, 0)

            @pl.when(s % 2 == 1)
            def _():
                gather_round(s, 1)

        def body(s, carry):
            @pl.when(s >= 2)
            def _():
                @pl.when(s % 2 == 0)
                def _():
                    wait_round(0)

                @pl.when(s % 2 == 1)
                def _():
                    wait_round(1)

            round_with_parity(s)
            return carry

        lax.fori_loop(0, _NSUB, body, 0)
        wait_round(1)
        wait_round(0)

    return k


_gather_kernel = _make_kernel()


def kernel(relative_position_bias_table, relative_position_index):
    table_flat = relative_position_bias_table.T.reshape(-1)
    idx_flat = relative_position_index.reshape(-1).astype(jnp.int32)
    out = _gather_kernel(table_flat, idx_flat)
    n0, n1 = relative_position_index.shape
    return out.reshape(_H, n0, n1)
